# baseline (speedup 1.0000x reference)
import functools

import jax
import jax.numpy as jnp
from jax import lax
from jax.experimental import pallas as pl
from jax.experimental.pallas import tpu as pltpu
from jax.experimental.pallas import tpu_sc as plsc

N_NODES = 10240
FEAT = 128
HALF = 64
N_EDGES = 327680
N_ROWS = 80
NC = 2
NS = 16
NW = NC * NS
CHUNK = 128
EPT = N_EDGES // NW
NCHUNK = EPT // CHUNK
STRIPE = N_NODES // NS


@functools.cache
def _mesh():
    return plsc.VectorSubcoreMesh(
        core_axis_name="c", subcore_axis_name="s", num_cores=NC, num_subcores=NS
    )


def _deg_body(dst_hbm, ones_hbm, zeros_hbm, deg_hbm, dst_v, ones_v, deg_sh, dsem):
    cid = lax.axis_index("c")
    sid = lax.axis_index("s")
    wid = sid * NC + cid
    row0 = pl.multiple_of(wid * NCHUNK, 8)
    pltpu.sync_copy(dst_hbm.at[pl.ds(row0, NCHUNK)], dst_v)
    pltpu.sync_copy(ones_hbm, ones_v)
    s0 = pl.multiple_of(sid * STRIPE, 8)
    pltpu.sync_copy(zeros_hbm.at[pl.ds(s0, STRIPE)], deg_sh.at[pl.ds(s0, STRIPE)])
    plsc.subcore_barrier()

    @pl.loop(0, NCHUNK)
    def _(j):
        pltpu.async_copy(ones_v, deg_sh.at[dst_v.at[j]], dsem, add=True)

    @pl.loop(0, NCHUNK)
    def _(j):
        pltpu.make_async_copy(ones_v, deg_sh.at[dst_v.at[j]], dsem).wait()

    plsc.subcore_barrier()
    pltpu.sync_copy(deg_sh.at[pl.ds(s0, STRIPE)], deg_hbm.at[cid, pl.ds(s0, STRIPE)])


@functools.cache
def _deg_call():
    return pl.kernel(
        _deg_body,
        out_type=jax.ShapeDtypeStruct((NC, N_NODES), jnp.float32),
        mesh=_mesh(),
        scratch_types=[
            pltpu.VMEM((NCHUNK, CHUNK), jnp.int32),
            pltpu.VMEM((CHUNK,), jnp.float32),
            pltpu.VMEM_SHARED((N_NODES,), jnp.float32),
            pltpu.SemaphoreType.DMA,
        ],
    )


NBUF = 2
DHALF = NCHUNK // 2


def _msg_body(y_hbm, src_hbm, dst_hbm, zeros_hbm, acc_hbm,
              src_v, dst_v, rows, gsems, ssems, acc_sh):
    cid = lax.axis_index("c")
    sid = lax.axis_index("s")
    wid = sid * NC + cid
    row0 = pl.multiple_of(wid * NCHUNK, 8)
    pltpu.sync_copy(src_hbm.at[pl.ds(row0, NCHUNK)], src_v)
    pltpu.sync_copy(dst_hbm.at[pl.ds(row0, DHALF)], dst_v)
    s0 = pl.multiple_of(sid * STRIPE, 8)

    for b in range(NBUF):
        pltpu.async_copy(y_hbm.at[src_v.at[b]], rows[b], gsems[b])

    @pl.when(cid == 0)
    def _():
        pltpu.sync_copy(y_hbm.at[pl.ds(s0, STRIPE)], acc_sh.at[pl.ds(s0, STRIPE)])

    @pl.when(cid == 1)
    def _():
        pltpu.sync_copy(zeros_hbm.at[pl.ds(s0, STRIPE)], acc_sh.at[pl.ds(s0, STRIPE)])

    plsc.subcore_barrier()

    @pl.loop(0, NCHUNK, step=NBUF)
    def _(j):
        @pl.when(j == DHALF)
        def _():
            pltpu.sync_copy(dst_hbm.at[pl.ds(row0 + DHALF, DHALF)], dst_v)

        jm = lax.rem(j, DHALF)
        for b in range(NBUF):
            pltpu.make_async_copy(y_hbm.at[src_v.at[j + b]], rows[b],
                                  gsems[b]).wait()
            pltpu.async_copy(rows[b], acc_sh.at[dst_v.at[jm + b]], ssems[b],
                             add=True)
        for b in range(NBUF):
            nxt = j + NBUF + b

            @pl.when(nxt < NCHUNK)
            def _():
                pltpu.make_async_copy(rows[b], acc_sh.at[dst_v.at[0]],
                                      ssems[b]).wait()
                pltpu.async_copy(y_hbm.at[src_v.at[nxt]], rows[b], gsems[b])

    for b in range(NBUF):
        pltpu.make_async_copy(rows[b], acc_sh.at[dst_v.at[0]], ssems[b]).wait()
    plsc.subcore_barrier()
    pltpu.sync_copy(acc_sh.at[pl.ds(s0, STRIPE)],
                    acc_hbm.at[cid, pl.ds(s0, STRIPE)])


@functools.cache
def _msg_call():
    return pl.kernel(
        _msg_body,
        out_type=jax.ShapeDtypeStruct((NC, N_NODES, FEAT), jnp.float32),
        mesh=_mesh(),
        scratch_types=[
            pltpu.VMEM((NCHUNK, CHUNK), jnp.int32),
            pltpu.VMEM((DHALF, CHUNK), jnp.int32),
            [pltpu.VMEM((CHUNK, FEAT), jnp.float32) for _ in range(NBUF)],
            [pltpu.SemaphoreType.DMA for _ in range(NBUF)],
            [pltpu.SemaphoreType.DMA for _ in range(NBUF)],
            pltpu.VMEM_SHARED((N_NODES, FEAT), jnp.float32),
        ],
        compiler_params=pltpu.CompilerParams(use_tc_tiling_on_sc=False),
    )


def _scale_body(deg_ref, x_ref, wg_ref, y_ref, dinv_ref):
    d = deg_ref[0] + deg_ref[1] + 1.0
    di = lax.rsqrt(d)
    dinv_ref[...] = jnp.broadcast_to(di, (di.shape[0], 8))
    xw = jnp.dot(x_ref[...], wg_ref[...], preferred_element_type=jnp.float32)
    y_ref[...] = xw * di


def _scale(deg, x, Wg):
    B = N_NODES // 8
    return pl.pallas_call(
        _scale_body,
        grid=(8,),
        in_specs=[
            pl.BlockSpec((NC, B, 1), lambda i: (0, i, 0)),
            pl.BlockSpec((B, FEAT), lambda i: (i, 0)),
            pl.BlockSpec((FEAT, FEAT), lambda i: (0, 0)),
        ],
        out_specs=[
            pl.BlockSpec((B, FEAT), lambda i: (i, 0)),
            pl.BlockSpec((B, 8), lambda i: (i, 0)),
        ],
        out_shape=[
            jax.ShapeDtypeStruct((N_NODES, FEAT), jnp.float32),
            jax.ShapeDtypeStruct((N_NODES, 8), jnp.float32),
        ],
    )(deg.reshape(NC, N_NODES, 1), x, Wg)


def _head_body(acc_ref, dinv_ref, bg_ref, w1_ref, b1_ref,
               w2_ref, b2_ref, o_ref, part_s):
    j = pl.program_id(0)
    di = dinv_ref[..., 0:1]
    g = (acc_ref[0] + acc_ref[1]) * di + bg_ref[...]
    h = jnp.where(g >= 0, g, 0.01 * g)
    part = jnp.dot(h[:, 0, :], w1_ref[0], preferred_element_type=jnp.float32)
    for m in range(1, 16):
        part += jnp.dot(h[:, m, :], w1_ref[m], preferred_element_type=jnp.float32)

    @pl.when(j == 0)
    def _():
        part_s[...] = part

    @pl.when(j > 0)
    def _():
        part_s[...] += part

    @pl.when(j == 7)
    def _():
        t = part_s[...] + b1_ref[...]
        t = jnp.where(t >= 0, t, 0.01 * t)
        o_ref[...] = jnp.dot(t, w2_ref[...], preferred_element_type=jnp.float32) + b2_ref[...]


def _head(acc, dinv, bg, W1, b1, W2, b2):
    return pl.pallas_call(
        _head_body,
        grid=(8,),
        in_specs=[
            pl.BlockSpec((NC, N_ROWS, 16, FEAT), lambda j: (0, 0, j, 0)),
            pl.BlockSpec((N_ROWS, 16, 8), lambda j: (0, j, 0)),
            pl.BlockSpec((1, 1, FEAT), lambda j: (0, 0, 0)),
            pl.BlockSpec((16, FEAT, FEAT), lambda j: (j, 0, 0)),
            pl.BlockSpec((1, FEAT), lambda j: (0, 0)),
            pl.BlockSpec((FEAT, 64), lambda j: (0, 0)),
            pl.BlockSpec((1, 64), lambda j: (0, 0)),
        ],
        out_specs=pl.BlockSpec((N_ROWS, 64), lambda j: (0, 0)),
        out_shape=jax.ShapeDtypeStruct((N_ROWS, 64), jnp.float32),
        scratch_shapes=[pltpu.VMEM((N_ROWS, FEAT), jnp.float32)],
    )(
        acc.reshape(NC, N_ROWS, FEAT, FEAT),
        dinv.reshape(N_ROWS, FEAT, 8),
        bg.reshape(1, 1, FEAT),
        W1.reshape(FEAT, FEAT, FEAT),
        b1.reshape(1, FEAT),
        W2,
        b2.reshape(1, 64),
    )


def kernel(x, edge_index, Wg, bg, W1, b1, W2, b2):
    src2 = edge_index[0].reshape(NW * NCHUNK, CHUNK)
    dst2 = edge_index[1].reshape(NW * NCHUNK, CHUNK)
    ones_c = jnp.ones((CHUNK,), jnp.float32)
    zeros_n = jnp.zeros((N_NODES,), jnp.float32)
    zeros_nf = jnp.zeros((N_NODES, FEAT), jnp.float32)

    deg = _deg_call()(dst2, ones_c, zeros_n)
    y, dinv = _scale(deg, x, Wg)
    acc = _msg_call()(y, src2, dst2, zeros_nf)
    return _head(acc, dinv, bg, W1, b1, W2, b2)


# speedup vs baseline: 1.1637x; 1.1637x over previous
import functools

import jax
import jax.numpy as jnp
from jax import lax
from jax.experimental import pallas as pl
from jax.experimental.pallas import tpu as pltpu
from jax.experimental.pallas import tpu_sc as plsc

N_NODES = 10240
FEAT = 128
HALF = 64
N_EDGES = 327680
N_ROWS = 80
NC = 2
NS = 16
NW = NC * NS
CHUNK = 128
EPT = N_EDGES // NW
NCHUNK = EPT // CHUNK
STRIPE = N_NODES // NS


@functools.cache
def _mesh():
    return plsc.VectorSubcoreMesh(
        core_axis_name="c", subcore_axis_name="s", num_cores=NC, num_subcores=NS
    )


def _deg_body(dst_hbm, ones_hbm, zeros_hbm, deg_hbm, dst_v, ones_v, deg_sh, dsem):
    cid = lax.axis_index("c")
    sid = lax.axis_index("s")
    wid = sid * NC + cid
    row0 = pl.multiple_of(wid * NCHUNK, 8)
    pltpu.sync_copy(dst_hbm.at[pl.ds(row0, NCHUNK)], dst_v)
    pltpu.sync_copy(ones_hbm, ones_v)
    s0 = pl.multiple_of(sid * STRIPE, 8)
    pltpu.sync_copy(zeros_hbm.at[pl.ds(s0, STRIPE)], deg_sh.at[pl.ds(s0, STRIPE)])
    plsc.subcore_barrier()

    @pl.loop(0, NCHUNK)
    def _(j):
        pltpu.async_copy(ones_v, deg_sh.at[dst_v.at[j]], dsem, add=True)

    @pl.loop(0, NCHUNK)
    def _(j):
        pltpu.make_async_copy(ones_v, deg_sh.at[dst_v.at[j]], dsem).wait()

    plsc.subcore_barrier()
    pltpu.sync_copy(deg_sh.at[pl.ds(s0, STRIPE)], deg_hbm.at[cid, pl.ds(s0, STRIPE)])


@functools.cache
def _deg_call():
    return pl.kernel(
        _deg_body,
        out_type=jax.ShapeDtypeStruct((NC, N_NODES), jnp.float32),
        mesh=_mesh(),
        scratch_types=[
            pltpu.VMEM((NCHUNK, CHUNK), jnp.int32),
            pltpu.VMEM((CHUNK,), jnp.float32),
            pltpu.VMEM_SHARED((N_NODES,), jnp.float32),
            pltpu.SemaphoreType.DMA,
        ],
    )


NBUF = 8


def _msg_body(y0_hbm, y1_hbm, src_hbm, dst_hbm, zeros_hbm, acc_hbm,
              src_v, dst_v, rows, gsems, ssems, acc_sh):
    cid = lax.axis_index("c")
    sid = lax.axis_index("s")
    wid = sid * NC + cid
    row0 = pl.multiple_of(wid * NCHUNK, 8)
    pltpu.sync_copy(src_hbm.at[pl.ds(row0, NCHUNK)], src_v)
    pltpu.sync_copy(dst_hbm.at[pl.ds(row0, NCHUNK)], dst_v)
    s0 = pl.multiple_of(sid * STRIPE, 8)

    for h, yv in ((0, y0_hbm), (1, y1_hbm)):
        for b in range(NBUF):
            pltpu.async_copy(yv.at[src_v.at[b]], rows[b], gsems[b])

        @pl.when(cid == 0)
        def _():
            pltpu.sync_copy(yv.at[pl.ds(s0, STRIPE)], acc_sh.at[pl.ds(s0, STRIPE)])

        @pl.when(cid == 1)
        def _():
            pltpu.sync_copy(zeros_hbm.at[pl.ds(s0, STRIPE)], acc_sh.at[pl.ds(s0, STRIPE)])

        plsc.subcore_barrier()

        @pl.loop(0, NCHUNK, step=NBUF)
        def _(j):
            for b in range(NBUF):
                pltpu.make_async_copy(yv.at[src_v.at[j + b]], rows[b],
                                      gsems[b]).wait()
                pltpu.async_copy(rows[b], acc_sh.at[dst_v.at[j + b]], ssems[b],
                                 add=True)
            for b in range(NBUF):
                nxt = j + NBUF + b

                @pl.when(nxt < NCHUNK)
                def _():
                    pltpu.make_async_copy(rows[b], acc_sh.at[dst_v.at[j + b]],
                                          ssems[b]).wait()
                    pltpu.async_copy(yv.at[src_v.at[nxt]], rows[b], gsems[b])

        for b in range(NBUF):
            pltpu.make_async_copy(rows[b], acc_sh.at[dst_v.at[NCHUNK - NBUF + b]],
                                  ssems[b]).wait()
        plsc.subcore_barrier()
        pltpu.sync_copy(acc_sh.at[pl.ds(s0, STRIPE)],
                        acc_hbm.at[cid, pl.ds(s0, STRIPE), pl.ds(HALF * h, HALF)])
        plsc.subcore_barrier()


@functools.cache
def _msg_call():
    return pl.kernel(
        _msg_body,
        out_type=jax.ShapeDtypeStruct((NC, N_NODES, FEAT), jnp.float32),
        mesh=_mesh(),
        scratch_types=[
            pltpu.VMEM((NCHUNK, CHUNK), jnp.int32),
            pltpu.VMEM((NCHUNK, CHUNK), jnp.int32),
            [pltpu.VMEM((CHUNK, HALF), jnp.float32) for _ in range(NBUF)],
            [pltpu.SemaphoreType.DMA for _ in range(NBUF)],
            [pltpu.SemaphoreType.DMA for _ in range(NBUF)],
            pltpu.VMEM_SHARED((N_NODES, HALF), jnp.float32),
        ],
        compiler_params=pltpu.CompilerParams(use_tc_tiling_on_sc=False),
    )


def _scale_body(deg_ref, x_ref, wg_ref, y0_ref, y1_ref, dinv_ref):
    d = deg_ref[0] + deg_ref[1] + 1.0
    di = lax.rsqrt(d)
    dinv_ref[...] = jnp.broadcast_to(di, (di.shape[0], 8))
    xw = jnp.dot(x_ref[...], wg_ref[...], preferred_element_type=jnp.float32)
    y = xw * di
    y0_ref[...] = y[:, :HALF]
    y1_ref[...] = y[:, HALF:]


def _scale(deg, x, Wg):
    B = N_NODES // 8
    return pl.pallas_call(
        _scale_body,
        grid=(8,),
        in_specs=[
            pl.BlockSpec((NC, B, 1), lambda i: (0, i, 0)),
            pl.BlockSpec((B, FEAT), lambda i: (i, 0)),
            pl.BlockSpec((FEAT, FEAT), lambda i: (0, 0)),
        ],
        out_specs=[
            pl.BlockSpec((B, HALF), lambda i: (i, 0)),
            pl.BlockSpec((B, HALF), lambda i: (i, 0)),
            pl.BlockSpec((B, 8), lambda i: (i, 0)),
        ],
        out_shape=[
            jax.ShapeDtypeStruct((N_NODES, HALF), jnp.float32),
            jax.ShapeDtypeStruct((N_NODES, HALF), jnp.float32),
            jax.ShapeDtypeStruct((N_NODES, 8), jnp.float32),
        ],
    )(deg.reshape(NC, N_NODES, 1), x, Wg)


def _head_body(acc_ref, dinv_ref, bg_ref, w1_ref, b1_ref,
               w2_ref, b2_ref, o_ref, part_s):
    j = pl.program_id(0)
    di = dinv_ref[..., 0:1]
    g = (acc_ref[0] + acc_ref[1]) * di + bg_ref[...]
    h = jnp.where(g >= 0, g, 0.01 * g)
    part = jnp.dot(h[:, 0, :], w1_ref[0], preferred_element_type=jnp.float32)
    for m in range(1, 16):
        part += jnp.dot(h[:, m, :], w1_ref[m], preferred_element_type=jnp.float32)

    @pl.when(j == 0)
    def _():
        part_s[...] = part

    @pl.when(j > 0)
    def _():
        part_s[...] += part

    @pl.when(j == 7)
    def _():
        t = part_s[...] + b1_ref[...]
        t = jnp.where(t >= 0, t, 0.01 * t)
        o_ref[...] = jnp.dot(t, w2_ref[...], preferred_element_type=jnp.float32) + b2_ref[...]


def _head(acc, dinv, bg, W1, b1, W2, b2):
    return pl.pallas_call(
        _head_body,
        grid=(8,),
        in_specs=[
            pl.BlockSpec((NC, N_ROWS, 16, FEAT), lambda j: (0, 0, j, 0)),
            pl.BlockSpec((N_ROWS, 16, 8), lambda j: (0, j, 0)),
            pl.BlockSpec((1, 1, FEAT), lambda j: (0, 0, 0)),
            pl.BlockSpec((16, FEAT, FEAT), lambda j: (j, 0, 0)),
            pl.BlockSpec((1, FEAT), lambda j: (0, 0)),
            pl.BlockSpec((FEAT, 64), lambda j: (0, 0)),
            pl.BlockSpec((1, 64), lambda j: (0, 0)),
        ],
        out_specs=pl.BlockSpec((N_ROWS, 64), lambda j: (0, 0)),
        out_shape=jax.ShapeDtypeStruct((N_ROWS, 64), jnp.float32),
        scratch_shapes=[pltpu.VMEM((N_ROWS, FEAT), jnp.float32)],
    )(
        acc.reshape(NC, N_ROWS, FEAT, FEAT),
        dinv.reshape(N_ROWS, FEAT, 8),
        bg.reshape(1, 1, FEAT),
        W1.reshape(FEAT, FEAT, FEAT),
        b1.reshape(1, FEAT),
        W2,
        b2.reshape(1, 64),
    )


def kernel(x, edge_index, Wg, bg, W1, b1, W2, b2):
    src2 = edge_index[0].reshape(NW * NCHUNK, CHUNK)
    dst2 = edge_index[1].reshape(NW * NCHUNK, CHUNK)
    ones_c = jnp.ones((CHUNK,), jnp.float32)
    zeros_n = jnp.zeros((N_NODES,), jnp.float32)
    zeros_nh = jnp.zeros((N_NODES, HALF), jnp.float32)

    deg = _deg_call()(dst2, ones_c, zeros_n)
    y0, y1, dinv = _scale(deg, x, Wg)
    acc = _msg_call()(y0, y1, src2, dst2, zeros_nh)
    return _head(acc, dinv, bg, W1, b1, W2, b2)


# speedup vs baseline: 1.2276x; 1.0549x over previous
import functools

import jax
import jax.numpy as jnp
from jax import lax
from jax.experimental import pallas as pl
from jax.experimental.pallas import tpu as pltpu
from jax.experimental.pallas import tpu_sc as plsc

N_NODES = 10240
FEAT = 128
HALF = 64
N_EDGES = 327680
N_ROWS = 80
NC = 2
NS = 16
NW = NC * NS
CHUNK = 128
EPT = N_EDGES // NW
NCHUNK = EPT // CHUNK
STRIPE = N_NODES // NS


@functools.cache
def _mesh():
    return plsc.VectorSubcoreMesh(
        core_axis_name="c", subcore_axis_name="s", num_cores=NC, num_subcores=NS
    )


def _deg_body(dst_hbm, ones_hbm, zeros_hbm, deg_hbm, dst_v, ones_v, deg_sh, dsem):
    cid = lax.axis_index("c")
    sid = lax.axis_index("s")
    wid = sid * NC + cid
    row0 = pl.multiple_of(wid * NCHUNK, 8)
    pltpu.sync_copy(dst_hbm.at[pl.ds(row0, NCHUNK)], dst_v)
    pltpu.sync_copy(ones_hbm, ones_v)
    s0 = pl.multiple_of(sid * STRIPE, 8)
    pltpu.sync_copy(zeros_hbm.at[pl.ds(s0, STRIPE)], deg_sh.at[pl.ds(s0, STRIPE)])
    plsc.subcore_barrier()

    @pl.loop(0, NCHUNK)
    def _(j):
        pltpu.async_copy(ones_v, deg_sh.at[dst_v.at[j]], dsem, add=True)

    @pl.loop(0, NCHUNK)
    def _(j):
        pltpu.make_async_copy(ones_v, deg_sh.at[dst_v.at[j]], dsem).wait()

    plsc.subcore_barrier()
    pltpu.sync_copy(deg_sh.at[pl.ds(s0, STRIPE)], deg_hbm.at[cid, pl.ds(s0, STRIPE)])


@functools.cache
def _deg_call():
    return pl.kernel(
        _deg_body,
        out_type=jax.ShapeDtypeStruct((NC, N_NODES), jnp.float32),
        mesh=_mesh(),
        scratch_types=[
            pltpu.VMEM((NCHUNK, CHUNK), jnp.int32),
            pltpu.VMEM((CHUNK,), jnp.float32),
            pltpu.VMEM_SHARED((N_NODES,), jnp.float32),
            pltpu.SemaphoreType.DMA,
        ],
    )


NBUF = 8


def _msg_body(y0_hbm, y1_hbm, src_hbm, dst_hbm, zeros_hbm, acc_hbm,
              src_v, dst_v, rows, gsems, ssems, acc_sh):
    cid = lax.axis_index("c")
    sid = lax.axis_index("s")
    wid = sid * NC + cid
    row0 = pl.multiple_of(wid * NCHUNK, 8)
    pltpu.sync_copy(src_hbm.at[pl.ds(row0, NCHUNK)], src_v)
    pltpu.sync_copy(dst_hbm.at[pl.ds(row0, NCHUNK)], dst_v)
    s0 = pl.multiple_of(sid * STRIPE, 8)

    for h, yv in ((0, y0_hbm), (1, y1_hbm)):
        for b in range(NBUF):
            pltpu.async_copy(yv.at[src_v.at[b]], rows[b], gsems[b])

        @pl.when(cid == 0)
        def _():
            pltpu.sync_copy(yv.at[pl.ds(s0, STRIPE)], acc_sh.at[pl.ds(s0, STRIPE)])

        @pl.when(cid == 1)
        def _():
            pltpu.sync_copy(zeros_hbm.at[pl.ds(s0, STRIPE)], acc_sh.at[pl.ds(s0, STRIPE)])

        plsc.subcore_barrier()

        @pl.loop(0, NCHUNK, step=NBUF)
        def _(j):
            for b in range(NBUF):
                pltpu.make_async_copy(yv.at[src_v.at[j + b]], rows[b],
                                      gsems[b]).wait()
                pltpu.async_copy(rows[b], acc_sh.at[dst_v.at[j + b]], ssems[b],
                                 add=True)
            for b in range(NBUF):
                nxt = j + NBUF + b

                @pl.when(nxt < NCHUNK)
                def _():
                    pltpu.make_async_copy(rows[b], acc_sh.at[dst_v.at[j + b]],
                                          ssems[b]).wait()
                    pltpu.async_copy(yv.at[src_v.at[nxt]], rows[b], gsems[b])

        for b in range(NBUF):
            pltpu.make_async_copy(rows[b], acc_sh.at[dst_v.at[NCHUNK - NBUF + b]],
                                  ssems[b]).wait()
        plsc.subcore_barrier()
        pltpu.sync_copy(acc_sh.at[pl.ds(s0, STRIPE)],
                        acc_hbm.at[cid, pl.ds(s0, STRIPE), pl.ds(HALF * h, HALF)])
        plsc.subcore_barrier()


@functools.cache
def _msg_call():
    return pl.kernel(
        _msg_body,
        out_type=jax.ShapeDtypeStruct((NC, N_NODES, FEAT), jnp.float32),
        mesh=_mesh(),
        scratch_types=[
            pltpu.VMEM((NCHUNK, CHUNK), jnp.int32),
            pltpu.VMEM((NCHUNK, CHUNK), jnp.int32),
            [pltpu.VMEM((CHUNK, HALF), jnp.float32) for _ in range(NBUF)],
            [pltpu.SemaphoreType.DMA for _ in range(NBUF)],
            [pltpu.SemaphoreType.DMA for _ in range(NBUF)],
            pltpu.VMEM_SHARED((N_NODES, HALF), jnp.float32),
        ],
        compiler_params=pltpu.CompilerParams(use_tc_tiling_on_sc=False),
    )


def _scale_body(deg_ref, x_ref, wg_ref, y0_ref, y1_ref, dinv_ref):
    d = deg_ref[0] + deg_ref[1] + 1.0
    di = lax.rsqrt(d).reshape(d.shape[0], 1)
    dinv_ref[...] = jnp.broadcast_to(di, (di.shape[0], 8))
    xw = jnp.dot(x_ref[...], wg_ref[...], preferred_element_type=jnp.float32)
    y = xw * di
    y0_ref[...] = y[:, :HALF]
    y1_ref[...] = y[:, HALF:]


def _scale(deg, x, Wg):
    B = N_NODES // 8
    return pl.pallas_call(
        _scale_body,
        grid=(8,),
        in_specs=[
            pl.BlockSpec((NC, B), lambda i: (0, i)),
            pl.BlockSpec((B, FEAT), lambda i: (i, 0)),
            pl.BlockSpec((FEAT, FEAT), lambda i: (0, 0)),
        ],
        out_specs=[
            pl.BlockSpec((B, HALF), lambda i: (i, 0)),
            pl.BlockSpec((B, HALF), lambda i: (i, 0)),
            pl.BlockSpec((B, 8), lambda i: (i, 0)),
        ],
        out_shape=[
            jax.ShapeDtypeStruct((N_NODES, HALF), jnp.float32),
            jax.ShapeDtypeStruct((N_NODES, HALF), jnp.float32),
            jax.ShapeDtypeStruct((N_NODES, 8), jnp.float32),
        ],
    )(deg, x, Wg)


def _head_body(acc_ref, dinv_ref, bg_ref, w1_ref, b1_ref,
               w2_ref, b2_ref, o_ref, part_s):
    j = pl.program_id(0)
    di = dinv_ref[..., 0:1]
    g = (acc_ref[0] + acc_ref[1]) * di + bg_ref[...]
    h = jnp.where(g >= 0, g, 0.01 * g)
    part = jnp.dot(h[:, 0, :], w1_ref[0], preferred_element_type=jnp.float32)
    for m in range(1, 16):
        part += jnp.dot(h[:, m, :], w1_ref[m], preferred_element_type=jnp.float32)

    @pl.when(j == 0)
    def _():
        part_s[...] = part

    @pl.when(j > 0)
    def _():
        part_s[...] += part

    @pl.when(j == 7)
    def _():
        t = part_s[...] + b1_ref[...]
        t = jnp.where(t >= 0, t, 0.01 * t)
        o_ref[...] = jnp.dot(t, w2_ref[...], preferred_element_type=jnp.float32) + b2_ref[...]


def _head(acc, dinv, bg, W1, b1, W2, b2):
    return pl.pallas_call(
        _head_body,
        grid=(8,),
        in_specs=[
            pl.BlockSpec((NC, N_ROWS, 16, FEAT), lambda j: (0, 0, j, 0)),
            pl.BlockSpec((N_ROWS, 16, 8), lambda j: (0, j, 0)),
            pl.BlockSpec((1, 1, FEAT), lambda j: (0, 0, 0)),
            pl.BlockSpec((16, FEAT, FEAT), lambda j: (j, 0, 0)),
            pl.BlockSpec((1, FEAT), lambda j: (0, 0)),
            pl.BlockSpec((FEAT, 64), lambda j: (0, 0)),
            pl.BlockSpec((1, 64), lambda j: (0, 0)),
        ],
        out_specs=pl.BlockSpec((N_ROWS, 64), lambda j: (0, 0)),
        out_shape=jax.ShapeDtypeStruct((N_ROWS, 64), jnp.float32),
        scratch_shapes=[pltpu.VMEM((N_ROWS, FEAT), jnp.float32)],
    )(
        acc.reshape(NC, N_ROWS, FEAT, FEAT),
        dinv.reshape(N_ROWS, FEAT, 8),
        bg.reshape(1, 1, FEAT),
        W1.reshape(FEAT, FEAT, FEAT),
        b1.reshape(1, FEAT),
        W2,
        b2.reshape(1, 64),
    )


def kernel(x, edge_index, Wg, bg, W1, b1, W2, b2):
    src2 = edge_index[0].reshape(NW * NCHUNK, CHUNK)
    dst2 = edge_index[1].reshape(NW * NCHUNK, CHUNK)
    ones_c = jnp.ones((CHUNK,), jnp.float32)
    zeros_n = jnp.zeros((N_NODES,), jnp.float32)
    zeros_nh = jnp.zeros((N_NODES, HALF), jnp.float32)

    deg = _deg_call()(dst2, ones_c, zeros_n)
    y0, y1, dinv = _scale(deg, x, Wg)
    acc = _msg_call()(y0, y1, src2, dst2, zeros_nh)
    return _head(acc, dinv, bg, W1, b1, W2, b2)


# speedup vs baseline: 1.2873x; 1.0486x over previous
import functools

import jax
import jax.numpy as jnp
from jax import lax
from jax.experimental import pallas as pl
from jax.experimental.pallas import tpu as pltpu
from jax.experimental.pallas import tpu_sc as plsc

N_NODES = 10240
FEAT = 128
HALF = 64
N_EDGES = 327680
N_ROWS = 80
NC = 2
NS = 16
NW = NC * NS
CHUNK = 128
EPT = N_EDGES // NW
NCHUNK = EPT // CHUNK
STRIPE = N_NODES // NS


@functools.cache
def _mesh():
    return plsc.VectorSubcoreMesh(
        core_axis_name="c", subcore_axis_name="s", num_cores=NC, num_subcores=NS
    )


def _deg_body(dst_hbm, ones_hbm, zeros_hbm, deg_hbm, dst_v, ones_v, deg_sh, dsem):
    cid = lax.axis_index("c")
    sid = lax.axis_index("s")
    wid = sid * NC + cid
    row0 = pl.multiple_of(wid * NCHUNK, 8)
    pltpu.sync_copy(dst_hbm.at[pl.ds(row0, NCHUNK)], dst_v)
    pltpu.sync_copy(ones_hbm, ones_v)
    s0 = pl.multiple_of(sid * STRIPE, 8)
    pltpu.sync_copy(zeros_hbm.at[pl.ds(s0, STRIPE)], deg_sh.at[pl.ds(s0, STRIPE)])
    plsc.subcore_barrier()

    @pl.loop(0, NCHUNK)
    def _(j):
        pltpu.async_copy(ones_v, deg_sh.at[dst_v.at[j]], dsem, add=True)

    @pl.loop(0, NCHUNK)
    def _(j):
        pltpu.make_async_copy(ones_v, deg_sh.at[dst_v.at[j]], dsem).wait()

    plsc.subcore_barrier()
    pltpu.sync_copy(deg_sh.at[pl.ds(s0, STRIPE)], deg_hbm.at[cid, pl.ds(s0, STRIPE)])


@functools.cache
def _deg_call():
    return pl.kernel(
        _deg_body,
        out_type=jax.ShapeDtypeStruct((NC, N_NODES), jnp.float32),
        mesh=_mesh(),
        scratch_types=[
            pltpu.VMEM((NCHUNK, CHUNK), jnp.int32),
            pltpu.VMEM((CHUNK,), jnp.float32),
            pltpu.VMEM_SHARED((N_NODES,), jnp.float32),
            pltpu.SemaphoreType.DMA,
        ],
    )


NBUF = 8


def _msg_body(yv, src_hbm, dst_hbm, zeros_hbm, acc_hbm,
              src_v, dst_v, rows, gsems, ssems, acc_sh):
    cid = lax.axis_index("c")
    sid = lax.axis_index("s")
    wid = sid * NC + cid
    row0 = pl.multiple_of(wid * NCHUNK, 8)
    pltpu.sync_copy(src_hbm.at[pl.ds(row0, NCHUNK)], src_v)
    pltpu.sync_copy(dst_hbm.at[pl.ds(row0, NCHUNK)], dst_v)
    s0 = pl.multiple_of(sid * STRIPE, 8)

    @pl.loop(0, NCHUNK)
    def _(j):
        for c in range(CHUNK // 16):
            sl = pl.ds(16 * c, 16)
            src_v[j, sl] = src_v[j, sl] * 2

    for h in (0, 1):
        if h == 1:
            @pl.loop(0, NCHUNK)
            def _(j):
                for c in range(CHUNK // 16):
                    sl = pl.ds(16 * c, 16)
                    src_v[j, sl] = src_v[j, sl] + 1

        for b in range(NBUF):
            pltpu.async_copy(yv.at[src_v.at[b]], rows[b], gsems[b])

        pltpu.sync_copy(zeros_hbm.at[pl.ds(s0, STRIPE)], acc_sh.at[pl.ds(s0, STRIPE)])
        plsc.subcore_barrier()

        @pl.loop(0, NCHUNK, step=NBUF)
        def _(j):
            for b in range(NBUF):
                pltpu.make_async_copy(yv.at[src_v.at[j + b]], rows[b],
                                      gsems[b]).wait()
                pltpu.async_copy(rows[b], acc_sh.at[dst_v.at[j + b]], ssems[b],
                                 add=True)
            for b in range(NBUF):
                nxt = j + NBUF + b

                @pl.when(nxt < NCHUNK)
                def _():
                    pltpu.make_async_copy(rows[b], acc_sh.at[dst_v.at[j + b]],
                                          ssems[b]).wait()
                    pltpu.async_copy(yv.at[src_v.at[nxt]], rows[b], gsems[b])

        for b in range(NBUF):
            pltpu.make_async_copy(rows[b], acc_sh.at[dst_v.at[NCHUNK - NBUF + b]],
                                  ssems[b]).wait()
        plsc.subcore_barrier()
        pltpu.sync_copy(acc_sh.at[pl.ds(s0, STRIPE)],
                        acc_hbm.at[cid, pl.ds(s0, STRIPE), pl.ds(HALF * h, HALF)])
        plsc.subcore_barrier()


@functools.cache
def _msg_call():
    return pl.kernel(
        _msg_body,
        out_type=jax.ShapeDtypeStruct((NC, N_NODES, FEAT), jnp.float32),
        mesh=_mesh(),
        scratch_types=[
            pltpu.VMEM((NCHUNK, CHUNK), jnp.int32),
            pltpu.VMEM((NCHUNK, CHUNK), jnp.int32),
            [pltpu.VMEM((CHUNK, HALF), jnp.float32) for _ in range(NBUF)],
            [pltpu.SemaphoreType.DMA for _ in range(NBUF)],
            [pltpu.SemaphoreType.DMA for _ in range(NBUF)],
            pltpu.VMEM_SHARED((N_NODES, HALF), jnp.float32),
        ],
        compiler_params=pltpu.CompilerParams(use_tc_tiling_on_sc=False),
    )


def _scale_body(deg_ref, x_ref, wg_ref, y_ref, dinv_ref):
    d = deg_ref[0] + deg_ref[1] + 1.0
    di = lax.rsqrt(d).reshape(d.shape[0], 1)
    dinv_ref[...] = jnp.broadcast_to(di, (di.shape[0], 8))
    xw = jnp.dot(x_ref[...], wg_ref[...], preferred_element_type=jnp.float32)
    y_ref[...] = xw * di


def _scale(deg, x, Wg):
    B = N_NODES // 8
    return pl.pallas_call(
        _scale_body,
        grid=(8,),
        in_specs=[
            pl.BlockSpec((NC, B), lambda i: (0, i)),
            pl.BlockSpec((B, FEAT), lambda i: (i, 0)),
            pl.BlockSpec((FEAT, FEAT), lambda i: (0, 0)),
        ],
        out_specs=[
            pl.BlockSpec((B, FEAT), lambda i: (i, 0)),
            pl.BlockSpec((B, 8), lambda i: (i, 0)),
        ],
        out_shape=[
            jax.ShapeDtypeStruct((N_NODES, FEAT), jnp.float32),
            jax.ShapeDtypeStruct((N_NODES, 8), jnp.float32),
        ],
    )(deg, x, Wg)


def _head_body(acc_ref, y_ref, dinv_ref, bg_ref, w1_ref, b1_ref,
               w2_ref, b2_ref, o_ref, part_s):
    j = pl.program_id(0)
    di = dinv_ref[..., 0:1]
    g = (acc_ref[0] + acc_ref[1] + y_ref[...]) * di + bg_ref[...]
    h = jnp.where(g >= 0, g, 0.01 * g)
    part = jnp.dot(h[:, 0, :], w1_ref[0], preferred_element_type=jnp.float32)
    for m in range(1, 16):
        part += jnp.dot(h[:, m, :], w1_ref[m], preferred_element_type=jnp.float32)

    @pl.when(j == 0)
    def _():
        part_s[...] = part

    @pl.when(j > 0)
    def _():
        part_s[...] += part

    @pl.when(j == 7)
    def _():
        t = part_s[...] + b1_ref[...]
        t = jnp.where(t >= 0, t, 0.01 * t)
        o_ref[...] = jnp.dot(t, w2_ref[...], preferred_element_type=jnp.float32) + b2_ref[...]


def _head(acc, y, dinv, bg, W1, b1, W2, b2):
    return pl.pallas_call(
        _head_body,
        grid=(8,),
        in_specs=[
            pl.BlockSpec((NC, N_ROWS, 16, FEAT), lambda j: (0, 0, j, 0)),
            pl.BlockSpec((N_ROWS, 16, FEAT), lambda j: (0, j, 0)),
            pl.BlockSpec((N_ROWS, 16, 8), lambda j: (0, j, 0)),
            pl.BlockSpec((1, 1, FEAT), lambda j: (0, 0, 0)),
            pl.BlockSpec((16, FEAT, FEAT), lambda j: (j, 0, 0)),
            pl.BlockSpec((1, FEAT), lambda j: (0, 0)),
            pl.BlockSpec((FEAT, 64), lambda j: (0, 0)),
            pl.BlockSpec((1, 64), lambda j: (0, 0)),
        ],
        out_specs=pl.BlockSpec((N_ROWS, 64), lambda j: (0, 0)),
        out_shape=jax.ShapeDtypeStruct((N_ROWS, 64), jnp.float32),
        scratch_shapes=[pltpu.VMEM((N_ROWS, FEAT), jnp.float32)],
    )(
        acc.reshape(NC, N_ROWS, FEAT, FEAT),
        y.reshape(N_ROWS, FEAT, FEAT),
        dinv.reshape(N_ROWS, FEAT, 8),
        bg.reshape(1, 1, FEAT),
        W1.reshape(FEAT, FEAT, FEAT),
        b1.reshape(1, FEAT),
        W2,
        b2.reshape(1, 64),
    )


def kernel(x, edge_index, Wg, bg, W1, b1, W2, b2):
    src2 = edge_index[0].reshape(NW * NCHUNK, CHUNK)
    dst2 = edge_index[1].reshape(NW * NCHUNK, CHUNK)
    ones_c = jnp.ones((CHUNK,), jnp.float32)
    zeros_n = jnp.zeros((N_NODES,), jnp.float32)
    zeros_nh = jnp.zeros((N_NODES, HALF), jnp.float32)

    deg = _deg_call()(dst2, ones_c, zeros_n)
    y, dinv = _scale(deg, x, Wg)
    yv = y.reshape(2 * N_NODES, HALF)
    acc = _msg_call()(yv, src2, dst2, zeros_nh)
    return _head(acc, y, dinv, bg, W1, b1, W2, b2)
